# baseline (speedup 1.0000x reference)
import jax
import jax.numpy as jnp
from jax.experimental import pallas as pl
from jax.experimental.pallas import tpu as pltpu

HIDDEN = 768
INTER = 768
NUM_EXPERTS = 8
ALPHA = 1.702
LIMIT = 7.0


def _moe_dense_kernel(x_ref, rw_ref, rb_ref, gup_ref, gupb_ref,
                      dw_ref, db_ref, out_ref,
                      s0_ref, s1_ref, i0_ref, i1_ref):
    e = pl.program_id(0)

    @pl.when(e == 0)
    def _router():
        x = x_ref[...]
        logits = jax.lax.dot_general(
            x, rw_ref[...], (((1,), (1,)), ((), ())),
            preferred_element_type=jnp.float32)
        logits = logits + rb_ref[...]
        eids = jax.lax.broadcasted_iota(jnp.int32, logits.shape, 1)
        i0 = jnp.argmax(logits, axis=1)[:, None]
        v0 = jnp.max(logits, axis=1)[:, None]
        masked = jnp.where(eids == i0, -jnp.inf, logits)
        i1 = jnp.argmax(masked, axis=1)[:, None]
        v1 = jnp.max(masked, axis=1)[:, None]
        s0 = jax.nn.sigmoid(v0 - v1)
        s0_ref[...] = s0
        s1_ref[...] = 1.0 - s0
        i0_ref[...] = i0
        i1_ref[...] = i1
        out_ref[...] = jnp.zeros_like(out_ref)

    x = x_ref[...].astype(jnp.bfloat16)
    xg = jnp.dot(x, gup_ref[0].astype(jnp.bfloat16),
                 preferred_element_type=jnp.float32) + gupb_ref[0]
    up_sh = pltpu.roll(xg, shift=2 * INTER - 1, axis=1)
    gate = jnp.minimum(xg, LIMIT)
    up = jnp.clip(up_sh, -LIMIT, LIMIT)
    glu = gate * jax.nn.sigmoid(gate * ALPHA)
    h = ((up + 1.0) * glu).astype(jnp.bfloat16)
    j_ids = jax.lax.broadcasted_iota(jnp.int32, (2 * INTER, INTER), 0)
    k_ids = jax.lax.broadcasted_iota(jnp.int32, (2 * INTER, INTER), 1)
    sel = (j_ids == 2 * k_ids).astype(jnp.bfloat16)
    dw2 = jnp.dot(sel, dw_ref[0].astype(jnp.bfloat16),
                  preferred_element_type=jnp.float32).astype(jnp.bfloat16)
    y = jnp.dot(h, dw2, preferred_element_type=jnp.float32) + db_ref[0]
    w = s0_ref[...] * (i0_ref[...] == e).astype(jnp.float32) \
        + s1_ref[...] * (i1_ref[...] == e).astype(jnp.float32)
    out_ref[...] += w * y


def kernel(hidden_states, router_w, router_b, gate_up_w, gate_up_b, down_w, down_b):
    B, S, H = hidden_states.shape
    T = B * S
    x = hidden_states.reshape(T, H)
    E = NUM_EXPERTS
    F = INTER
    gupb = gate_up_b.reshape(E, 1, 2 * F)
    db = down_b.reshape(E, 1, H)
    rb = router_b.reshape(1, E)

    out = pl.pallas_call(
        _moe_dense_kernel,
        grid=(E,),
        in_specs=[
            pl.BlockSpec((T, H), lambda e: (0, 0)),
            pl.BlockSpec((E, H), lambda e: (0, 0)),
            pl.BlockSpec((1, E), lambda e: (0, 0)),
            pl.BlockSpec((1, H, 2 * F), lambda e: (e, 0, 0)),
            pl.BlockSpec((1, 1, 2 * F), lambda e: (e, 0, 0)),
            pl.BlockSpec((1, F, H), lambda e: (e, 0, 0)),
            pl.BlockSpec((1, 1, H), lambda e: (e, 0, 0)),
        ],
        out_specs=pl.BlockSpec((T, H), lambda e: (0, 0)),
        out_shape=jax.ShapeDtypeStruct((T, H), jnp.float32),
        scratch_shapes=[
            pltpu.VMEM((T, 1), jnp.float32),
            pltpu.VMEM((T, 1), jnp.float32),
            pltpu.VMEM((T, 1), jnp.int32),
            pltpu.VMEM((T, 1), jnp.int32),
        ],
        compiler_params=pltpu.CompilerParams(
            dimension_semantics=("arbitrary",),
        ),
    )(x, router_w, rb, gate_up_w, gupb, down_w, db)
    return out.reshape(B, S, H)


# speedup vs baseline: 1.0152x; 1.0152x over previous
import jax
import jax.numpy as jnp
from jax.experimental import pallas as pl
from jax.experimental.pallas import tpu as pltpu

HIDDEN = 768
INTER = 768
NUM_EXPERTS = 8
ALPHA = 1.702
LIMIT = 7.0


def _moe_dense_kernel(x_ref, rw_ref, rb_ref, gup_ref, gupb_ref,
                      dw_ref, db_ref, out_ref,
                      s0_ref, s1_ref, i0_ref, i1_ref):
    e = pl.program_id(0)

    @pl.when(e == 0)
    def _router():
        x = x_ref[...]
        logits = jax.lax.dot_general(
            x, rw_ref[...], (((1,), (1,)), ((), ())),
            preferred_element_type=jnp.float32)
        logits = logits + rb_ref[...]
        eids = jax.lax.broadcasted_iota(jnp.int32, logits.shape, 1)
        i0 = jnp.argmax(logits, axis=1)[:, None]
        v0 = jnp.max(logits, axis=1)[:, None]
        masked = jnp.where(eids == i0, -jnp.inf, logits)
        i1 = jnp.argmax(masked, axis=1)[:, None]
        v1 = jnp.max(masked, axis=1)[:, None]
        s0 = jax.nn.sigmoid(v0 - v1)
        s0_ref[...] = s0
        s1_ref[...] = 1.0 - s0
        i0_ref[...] = i0
        i1_ref[...] = i1
        out_ref[...] = jnp.zeros_like(out_ref)

    x = x_ref[...].astype(jnp.bfloat16)
    xg = jnp.dot(x, gup_ref[0].astype(jnp.bfloat16),
                 preferred_element_type=jnp.float32) + gupb_ref[0]
    gate = jnp.minimum(xg, LIMIT)
    glu = gate * jax.nn.sigmoid(gate * ALPHA)
    upc = (jnp.clip(xg, -LIMIT, LIMIT) + 1.0).astype(jnp.bfloat16)
    up_sh = pltpu.roll(upc, shift=2 * INTER - 1, axis=1)
    h = up_sh * glu.astype(jnp.bfloat16)
    j_ids = jax.lax.broadcasted_iota(jnp.int32, (2 * INTER, INTER), 0)
    k_ids = jax.lax.broadcasted_iota(jnp.int32, (2 * INTER, INTER), 1)
    sel = (j_ids == 2 * k_ids).astype(jnp.bfloat16)
    dw2 = jnp.dot(sel, dw_ref[0].astype(jnp.bfloat16),
                  preferred_element_type=jnp.float32).astype(jnp.bfloat16)
    y = jnp.dot(h, dw2, preferred_element_type=jnp.float32) + db_ref[0]
    w = s0_ref[...] * (i0_ref[...] == e).astype(jnp.float32) \
        + s1_ref[...] * (i1_ref[...] == e).astype(jnp.float32)
    out_ref[...] += w * y


def kernel(hidden_states, router_w, router_b, gate_up_w, gate_up_b, down_w, down_b):
    B, S, H = hidden_states.shape
    T = B * S
    x = hidden_states.reshape(T, H)
    E = NUM_EXPERTS
    F = INTER
    gupb = gate_up_b.reshape(E, 1, 2 * F)
    db = down_b.reshape(E, 1, H)
    rb = router_b.reshape(1, E)

    out = pl.pallas_call(
        _moe_dense_kernel,
        grid=(E,),
        in_specs=[
            pl.BlockSpec((T, H), lambda e: (0, 0)),
            pl.BlockSpec((E, H), lambda e: (0, 0)),
            pl.BlockSpec((1, E), lambda e: (0, 0)),
            pl.BlockSpec((1, H, 2 * F), lambda e: (e, 0, 0)),
            pl.BlockSpec((1, 1, 2 * F), lambda e: (e, 0, 0)),
            pl.BlockSpec((1, F, H), lambda e: (e, 0, 0)),
            pl.BlockSpec((1, 1, H), lambda e: (e, 0, 0)),
        ],
        out_specs=pl.BlockSpec((T, H), lambda e: (0, 0)),
        out_shape=jax.ShapeDtypeStruct((T, H), jnp.float32),
        scratch_shapes=[
            pltpu.VMEM((T, 1), jnp.float32),
            pltpu.VMEM((T, 1), jnp.float32),
            pltpu.VMEM((T, 1), jnp.int32),
            pltpu.VMEM((T, 1), jnp.int32),
        ],
        compiler_params=pltpu.CompilerParams(
            dimension_semantics=("arbitrary",),
        ),
    )(x, router_w, rb, gate_up_w, gupb, down_w, db)
    return out.reshape(B, S, H)
